# baseline (speedup 1.0000x reference)
import jax
import jax.numpy as jnp
from jax.experimental import pallas as pl
from jax.experimental.pallas import tpu as pltpu

_TILE_ROWS = 2048


def _fused_mlp_tile(xp_ref, wa_ref, ba_ref, wb_ref, bb_ref, out_ref):
    h = jax.lax.dot(xp_ref[...], wa_ref[...],
                    preferred_element_type=jnp.float32)
    h = jnp.maximum(h + ba_ref[...], 0.0)
    y = jax.lax.dot(h, wb_ref[...], preferred_element_type=jnp.float32)
    out_ref[...] = jnp.maximum(y + bb_ref[...], 0.0)


def kernel(x, w1, b1, w2, b2):
    B, N, D = x.shape
    H = w1.shape[0]
    pack = 128 // D
    M = B * N
    rows = -(-M // pack)

    xf = x.reshape(M, D)
    if rows * pack != M:
        xf = jnp.pad(xf, ((0, rows * pack - M), (0, 0)))
    xp = xf.reshape(rows, pack * D)

    wa = jnp.kron(jnp.eye(pack, dtype=x.dtype), w1.T)
    wb = jnp.kron(jnp.eye(pack, dtype=x.dtype), w2.T)
    ba = jnp.tile(b1, pack)[None, :]
    bb = jnp.tile(b2, pack)[None, :]

    tr = min(_TILE_ROWS, rows)
    grid = (pl.cdiv(rows, tr),)

    out = pl.pallas_call(
        _fused_mlp_tile,
        out_shape=jax.ShapeDtypeStruct((rows, pack), jnp.float32),
        grid=grid,
        in_specs=[
            pl.BlockSpec((tr, pack * D), lambda i: (i, 0)),
            pl.BlockSpec((pack * D, pack * H), lambda i: (0, 0)),
            pl.BlockSpec((1, pack * H), lambda i: (0, 0)),
            pl.BlockSpec((pack * H, pack), lambda i: (0, 0)),
            pl.BlockSpec((1, pack), lambda i: (0, 0)),
        ],
        out_specs=pl.BlockSpec((tr, pack), lambda i: (i, 0)),
        compiler_params=pltpu.CompilerParams(
            dimension_semantics=("parallel",),
            vmem_limit_bytes=64 * 1024 * 1024,
        ),
        cost_estimate=pl.CostEstimate(
            flops=(2 * D * H + 2 * H) * M,
            bytes_accessed=(D + 1) * 4 * M,
            transcendentals=0,
        ),
    )(xp, wa, ba, wb, bb)

    return out.reshape(-1)[:M].reshape(B, N)


# speedup vs baseline: 1.1151x; 1.1151x over previous
import jax
import jax.numpy as jnp
from jax.experimental import pallas as pl
from jax.experimental.pallas import tpu as pltpu

_BT = 8
_NT = 2048


def _decoder_tile(x_ref, w1_ref, b1c_ref, w2_ref, b2s_ref, o_ref):
    rows = []
    for b in range(x_ref.shape[0]):
        xt = x_ref[b].T
        h = jax.lax.dot_general(w1_ref[...], xt, (((1,), (0,)), ((), ())),
                                preferred_element_type=jnp.float32)
        h = jnp.maximum(h + b1c_ref[...], 0.0)
        y = jax.lax.dot_general(w2_ref[...], h, (((1,), (0,)), ((), ())),
                                preferred_element_type=jnp.float32)
        rows.append(jnp.maximum(y + b2s_ref[0], 0.0))
    o_ref[...] = jnp.concatenate(rows, axis=0)


def kernel(x, w1, b1, w2, b2):
    B, N, D = x.shape
    H = w1.shape[0]

    b1c = b1.reshape(H, 1)
    b2s = b2.reshape(1).astype(jnp.float32)

    grid = (B // _BT, pl.cdiv(N, _NT))

    out = pl.pallas_call(
        _decoder_tile,
        out_shape=jax.ShapeDtypeStruct((B, N), jnp.float32),
        grid=grid,
        in_specs=[
            pl.BlockSpec((_BT, _NT, D), lambda i, j: (i, j, 0)),
            pl.BlockSpec((H, D), lambda i, j: (0, 0)),
            pl.BlockSpec((H, 1), lambda i, j: (0, 0)),
            pl.BlockSpec((1, H), lambda i, j: (0, 0)),
            pl.BlockSpec(memory_space=pltpu.MemorySpace.SMEM),
        ],
        out_specs=pl.BlockSpec((_BT, _NT), lambda i, j: (i, j)),
        compiler_params=pltpu.CompilerParams(
            dimension_semantics=("parallel", "parallel"),
            vmem_limit_bytes=64 * 1024 * 1024,
        ),
        cost_estimate=pl.CostEstimate(
            flops=(2 * D * H + 2 * H) * B * N,
            bytes_accessed=(D + 1) * 4 * B * N,
            transcendentals=0,
        ),
    )(x, w1, b1c, w2, b2s)

    return out


# speedup vs baseline: 9.0559x; 8.1214x over previous
import jax
import jax.numpy as jnp
from jax.experimental import pallas as pl
from jax.experimental.pallas import tpu as pltpu

_BT = 8
_NT = 2048


def _decoder_tile(xt_ref, w1b_ref, b1b_ref, w2b_ref, b2s_ref, o_ref):
    h = jax.lax.dot_general(w1b_ref[...], xt_ref[...],
                            (((1,), (0,)), ((), ())),
                            preferred_element_type=jnp.float32)
    h = jnp.maximum(h + b1b_ref[...], 0.0)
    y = jax.lax.dot_general(w2b_ref[...], h,
                            (((1,), (0,)), ((), ())),
                            preferred_element_type=jnp.float32)
    o_ref[...] = jnp.maximum(y + b2s_ref[0], 0.0)


def kernel(x, w1, b1, w2, b2):
    B, N, D = x.shape
    H = w1.shape[0]

    xt = jnp.swapaxes(x, 1, 2).reshape(B * D, N)

    eye = jnp.eye(_BT, dtype=x.dtype)
    w1b = jnp.kron(eye, w1)
    w2b = jnp.kron(eye, w2)
    b1b = jnp.tile(b1, _BT).reshape(_BT * H, 1)
    b2s = b2.reshape(1).astype(jnp.float32)

    grid = (B // _BT, pl.cdiv(N, _NT))

    out = pl.pallas_call(
        _decoder_tile,
        out_shape=jax.ShapeDtypeStruct((B, N), jnp.float32),
        grid=grid,
        in_specs=[
            pl.BlockSpec((D * _BT, _NT), lambda i, j: (i, j)),
            pl.BlockSpec((H * _BT, D * _BT), lambda i, j: (0, 0)),
            pl.BlockSpec((H * _BT, 1), lambda i, j: (0, 0)),
            pl.BlockSpec((_BT, H * _BT), lambda i, j: (0, 0)),
            pl.BlockSpec(memory_space=pltpu.MemorySpace.SMEM),
        ],
        out_specs=pl.BlockSpec((_BT, _NT), lambda i, j: (i, j)),
        compiler_params=pltpu.CompilerParams(
            dimension_semantics=("parallel", "parallel"),
            vmem_limit_bytes=64 * 1024 * 1024,
        ),
        cost_estimate=pl.CostEstimate(
            flops=(2 * D * H + 2 * H) * B * N,
            bytes_accessed=(D + 1) * 4 * B * N,
            transcendentals=0,
        ),
    )(xt, w1b, b1b, w2b, b2s)

    return out


# speedup vs baseline: 20.0313x; 2.2120x over previous
import jax
import jax.numpy as jnp
from jax.experimental import pallas as pl
from jax.experimental.pallas import tpu as pltpu

_BT = 32
_NT = 4096


def _decoder_tile(xt_ref, w1b_ref, b1b_ref, w2b_ref, b2s_ref, o_ref):
    h = jax.lax.dot_general(w1b_ref[...], xt_ref[...],
                            (((1,), (0,)), ((), ())),
                            preferred_element_type=jnp.float32)
    h = jnp.maximum(h + b1b_ref[...], 0.0)
    y = jax.lax.dot_general(w2b_ref[...], h,
                            (((1,), (0,)), ((), ())),
                            preferred_element_type=jnp.float32)
    o_ref[...] = jnp.maximum(y + b2s_ref[0], 0.0)


def kernel(x, w1, b1, w2, b2):
    B, N, D = x.shape
    H = w1.shape[0]

    xt = jnp.swapaxes(x, 1, 2).reshape(B * D, N)

    eye = jnp.eye(_BT, dtype=x.dtype)
    w1b = jnp.kron(eye, w1)
    w2b = jnp.kron(eye, w2)
    b1b = jnp.tile(b1, _BT).reshape(_BT * H, 1)
    b2s = b2.reshape(1).astype(jnp.float32)

    grid = (B // _BT, pl.cdiv(N, _NT))

    out = pl.pallas_call(
        _decoder_tile,
        out_shape=jax.ShapeDtypeStruct((B, N), jnp.float32),
        grid=grid,
        in_specs=[
            pl.BlockSpec((D * _BT, _NT), lambda i, j: (i, j)),
            pl.BlockSpec((H * _BT, D * _BT), lambda i, j: (0, 0)),
            pl.BlockSpec((H * _BT, 1), lambda i, j: (0, 0)),
            pl.BlockSpec((_BT, H * _BT), lambda i, j: (0, 0)),
            pl.BlockSpec(memory_space=pltpu.MemorySpace.SMEM),
        ],
        out_specs=pl.BlockSpec((_BT, _NT), lambda i, j: (i, j)),
        compiler_params=pltpu.CompilerParams(
            dimension_semantics=("parallel", "parallel"),
            vmem_limit_bytes=64 * 1024 * 1024,
        ),
        cost_estimate=pl.CostEstimate(
            flops=(2 * D * H + 2 * H) * B * N,
            bytes_accessed=(D + 1) * 4 * B * N,
            transcendentals=0,
        ),
    )(xt, w1b, b1b, w2b, b2s)

    return out
